# baseline (speedup 1.0000x reference)
import functools

import jax
import jax.numpy as jnp
from jax import lax
from jax.experimental import pallas as pl
from jax.experimental.pallas import tpu as pltpu
from jax.experimental.pallas import tpu_sc as plsc


def _make_gather(NF, NW, NC, CR):
    r_per_w = NF // NW
    n_chunks = r_per_w // CR
    mesh = plsc.VectorSubcoreMesh(core_axis_name="c", subcore_axis_name="s")

    @functools.partial(
        pl.kernel,
        mesh=mesh,
        compiler_params=pltpu.CompilerParams(use_tc_tiling_on_sc=False),
        out_type=jax.ShapeDtypeStruct((NF, 128), jnp.float32),
        scratch_types=[
            pltpu.VMEM((r_per_w,), jnp.int32),
            pltpu.VMEM((CR, 128), jnp.float32),
            pltpu.SemaphoreType.DMA,
        ],
    )
    def k(idx_hbm, t128_hbm, out_hbm, idx_v, rows_v, sem):
        wid = lax.axis_index("s") * NC + lax.axis_index("c")
        base = wid * r_per_w
        pltpu.sync_copy(idx_hbm.at[pl.ds(wid * r_per_w, r_per_w)], idx_v)

        def body(c, carry):
            pltpu.async_copy(
                t128_hbm.at[idx_v.at[pl.ds(c * CR, CR)]], rows_v, sem
            ).wait()
            pltpu.sync_copy(rows_v, out_hbm.at[pl.ds(base + c * CR, CR)])
            return carry

        lax.fori_loop(0, n_chunks, body, 0)

    return k


def kernel(input_batch, table):
    B, H = input_batch.shape
    V, D = table.shape
    HP = 56

    info = plsc.get_sparse_core_info()
    NC, NS = info.num_cores, info.num_subcores
    NW = NC * NS
    CR = 512

    t128 = jnp.concatenate([table, table], axis=1)
    idxp = jnp.pad(input_batch.astype(jnp.int32), ((0, 0), (0, HP - H)))
    idxf = idxp.reshape(B * HP)
    out = _make_gather(B * HP, NW, NC, CR)(idxf, t128)
    return out.reshape(B, HP, 2 * D)[:, :H, :D]


# speedup vs baseline: 2.2938x; 2.2938x over previous
import functools

import jax
import jax.numpy as jnp
from jax import lax
from jax.experimental import pallas as pl
from jax.experimental.pallas import tpu as pltpu
from jax.experimental.pallas import tpu_sc as plsc


def _make_gather(NF, NW, NC, CR):
    r_per_w = NF // NW
    n_chunks = r_per_w // CR
    mesh = plsc.VectorSubcoreMesh(core_axis_name="c", subcore_axis_name="s")

    @functools.partial(
        pl.kernel,
        mesh=mesh,
        compiler_params=pltpu.CompilerParams(use_tc_tiling_on_sc=False),
        out_type=jax.ShapeDtypeStruct((NF, 128), jnp.float32),
        scratch_types=[
            pltpu.VMEM((r_per_w,), jnp.int32),
            pltpu.VMEM((CR, 128), jnp.float32),
            pltpu.SemaphoreType.DMA,
        ],
    )
    def k(idx_hbm, t128_hbm, out_hbm, idx_v, rows_v, sem):
        wid = lax.axis_index("s") * NC + lax.axis_index("c")
        base = wid * r_per_w
        pltpu.sync_copy(idx_hbm.at[pl.ds(wid * r_per_w, r_per_w)], idx_v)

        def body(c, carry):
            pltpu.async_copy(
                t128_hbm.at[idx_v.at[pl.ds(c * CR, CR)]], rows_v, sem
            ).wait()
            pltpu.sync_copy(rows_v, out_hbm.at[pl.ds(base + c * CR, CR)])
            return carry

        lax.fori_loop(0, n_chunks, body, 0)

    return k


def kernel(input_batch, table):
    B, H = input_batch.shape
    V, D = table.shape
    HP = 56

    info = plsc.get_sparse_core_info()
    NC, NS = info.num_cores, info.num_subcores
    NW = NC * NS
    CR = 512

    t128 = jnp.concatenate([table, table], axis=1)
    junk = (
        jnp.arange(B, dtype=jnp.int32)[:, None] * (HP - H)
        + jnp.arange(HP - H, dtype=jnp.int32)[None, :]
    )
    idxp = jnp.concatenate([input_batch.astype(jnp.int32), junk % V], axis=1)
    idxf = idxp.reshape(B * HP)
    out = _make_gather(B * HP, NW, NC, CR)(idxf, t128)
    return out.reshape(B, HP, 2 * D)[:, :H, :D]


# speedup vs baseline: 2.7942x; 1.2181x over previous
import functools

import jax
import jax.numpy as jnp
from jax import lax
from jax.experimental import pallas as pl
from jax.experimental.pallas import tpu as pltpu
from jax.experimental.pallas import tpu_sc as plsc


def _make_gather(NF, NW, NC, CR):
    r_per_w = NF // NW
    n_chunks = r_per_w // CR
    mesh = plsc.VectorSubcoreMesh(core_axis_name="c", subcore_axis_name="s")

    @functools.partial(
        pl.kernel,
        mesh=mesh,
        compiler_params=pltpu.CompilerParams(use_tc_tiling_on_sc=False),
        out_type=jax.ShapeDtypeStruct((NF, 128), jnp.float32),
        scratch_types=[
            pltpu.VMEM((r_per_w,), jnp.int32),
            pltpu.VMEM((CR, 128), jnp.float32),
            pltpu.SemaphoreType.DMA,
        ],
    )
    def k(idx_hbm, t128_hbm, out_hbm, idx_v, rows_v, sem):
        wid = lax.axis_index("s") * NC + lax.axis_index("c")
        base = wid * r_per_w
        pltpu.sync_copy(idx_hbm.at[pl.ds(wid * r_per_w, r_per_w)], idx_v)

        def body(c, carry):
            pltpu.async_copy(
                t128_hbm.at[idx_v.at[pl.ds(c * CR, CR)]], rows_v, sem
            ).wait()
            pltpu.sync_copy(rows_v, out_hbm.at[pl.ds(base + c * CR, CR)])
            return carry

        lax.fori_loop(0, n_chunks, body, 0)

    return k


def kernel(input_batch, table):
    B, H = input_batch.shape
    V, D = table.shape
    HP = 56

    info = plsc.get_sparse_core_info()
    NC, NS = info.num_cores, info.num_subcores
    NW = NC * NS
    CR = 512

    t128 = jnp.pad(table, ((0, 0), (0, D)))
    junk = (
        jnp.arange(B, dtype=jnp.int32)[:, None] * (HP - H)
        + jnp.arange(HP - H, dtype=jnp.int32)[None, :]
    )
    idxp = jnp.concatenate([input_batch.astype(jnp.int32), junk % V], axis=1)
    idxf = idxp.reshape(B * HP)
    out = _make_gather(B * HP, NW, NC, CR)(idxf, t128)
    return out.reshape(B, HP, 2 * D)[:, :H, :D]
